# baseline (speedup 1.0000x reference)
import functools

import jax
import jax.numpy as jnp
from jax import lax
from jax.experimental import pallas as pl
from jax.experimental.pallas import tpu as pltpu
from jax.experimental.pallas import tpu_sc as plsc

BATCH = 16384
SEQ = 200
EMB = 64
NC, NS = 2, 16
NW = NC * NS
BW = BATCH // NW
SBLK = 8
NBLK = SEQ // SBLK
BGRP = BW // 16
ROFF = 321

_MESH = plsc.VectorSubcoreMesh(core_axis_name="c", subcore_axis_name="s")


@functools.partial(
    pl.kernel,
    out_type=jax.ShapeDtypeStruct((SEQ, EMB, BATCH), jnp.float32),
    mesh=_MESH,
    scratch_types=[
        pltpu.VMEM((5 * EMB,), jnp.float32),
        pltpu.VMEM((16 * ROFF,), jnp.float32),
        pltpu.VMEM((SBLK, BW), jnp.int32),
        pltpu.VMEM((1, EMB, BW), jnp.float32),
        pltpu.VMEM((1, EMB, BW), jnp.float32),
        pltpu.SemaphoreType.DMA,
        pltpu.SemaphoreType.DMA,
    ],
    compiler_params=pltpu.CompilerParams(needs_layout_passes=False),
)
def _emb_lookup(x_hbm, tab_hbm, out_hbm, tab_v, tab_r, x_v, out_a, out_b,
                sem_a, sem_b):
    wid = lax.axis_index("s") * NC + lax.axis_index("c")
    b0 = wid * BW

    pltpu.sync_copy(tab_hbm, tab_v)
    lane = lax.iota(jnp.int32, 16)
    tl = [tab_v[pl.ds(16 * m, 16)] for m in range(5 * EMB // 16)]

    def repl(l, carry):
        for m in range(5 * EMB // 16):
            plsc.store_scatter(tab_r, [l * ROFF + 16 * m + lane], tl[m])
        return carry

    lax.fori_loop(0, 16, repl, 0)

    lane_off = lane * ROFF

    def build(row, out_v):
        @plsc.parallel_loop(0, BGRP, step=1, unroll=2)
        def bgrp(bb):
            boff = pl.multiple_of(bb * 16, 16)
            idxv = x_v[row, pl.ds(boff, 16)]
            addr = lane_off + idxv * EMB
            for e in range(EMB):
                out_v[0, e, pl.ds(boff, 16)] = plsc.load_gather(
                    tab_r, [addr + e])

    def pair(t, carry):
        s_a = 2 * t
        rem = lax.rem(s_a, SBLK)
        s0 = pl.multiple_of(s_a - rem, SBLK)

        @pl.when(rem == 0)
        def _():
            pltpu.sync_copy(x_hbm.at[pl.ds(s0, SBLK), pl.ds(b0, BW)], x_v)

        @pl.when(t > 0)
        def _():
            pltpu.make_async_copy(
                out_a, out_hbm.at[pl.ds(s_a - 2, 1), :, pl.ds(b0, BW)],
                sem_a).wait()

        build(rem, out_a)
        pltpu.make_async_copy(
            out_a, out_hbm.at[pl.ds(s_a, 1), :, pl.ds(b0, BW)], sem_a).start()

        @pl.when(t > 0)
        def _():
            pltpu.make_async_copy(
                out_b, out_hbm.at[pl.ds(s_a - 1, 1), :, pl.ds(b0, BW)],
                sem_b).wait()

        build(rem + 1, out_b)
        pltpu.make_async_copy(
            out_b, out_hbm.at[pl.ds(s_a + 1, 1), :, pl.ds(b0, BW)],
            sem_b).start()
        return carry

    lax.fori_loop(0, SEQ // 2, pair, 0)

    pltpu.make_async_copy(
        out_a, out_hbm.at[pl.ds(SEQ - 2, 1), :, pl.ds(b0, BW)], sem_a).wait()
    pltpu.make_async_copy(
        out_b, out_hbm.at[pl.ds(SEQ - 1, 1), :, pl.ds(b0, BW)], sem_b).wait()


def kernel(x, connectivity_embedding):
    xt = x.T
    tab1d = connectivity_embedding.reshape(-1)
    out_t = _emb_lookup(xt, tab1d)
    return lax.transpose(out_t, (2, 0, 1))


# speedup vs baseline: 2.4843x; 2.4843x over previous
import functools

import jax
import jax.numpy as jnp
from jax import lax
from jax.experimental import pallas as pl
from jax.experimental.pallas import tpu as pltpu
from jax.experimental.pallas import tpu_sc as plsc

BATCH = 16384
SEQ = 200
EMB = 64
NC, NS = 2, 16
NW = NC * NS
BW = BATCH // NW
SBLK = 8
NBLK = SEQ // SBLK
BGRP = BW // 16
ROFF = 321

_MESH = plsc.VectorSubcoreMesh(core_axis_name="c", subcore_axis_name="s")


@functools.partial(
    pl.kernel,
    out_type=jax.ShapeDtypeStruct((SEQ, EMB, BATCH), jnp.float32),
    mesh=_MESH,
    scratch_types=[
        pltpu.VMEM((5 * EMB,), jnp.float32),
        pltpu.VMEM((16 * ROFF,), jnp.float32),
        pltpu.VMEM((SBLK, BW), jnp.int32),
        pltpu.VMEM((3, EMB, BW), jnp.float32),
        pltpu.SemaphoreType.DMA,
        pltpu.SemaphoreType.DMA,
        pltpu.SemaphoreType.DMA,
    ],
    compiler_params=pltpu.CompilerParams(needs_layout_passes=False),
)
def _emb_lookup(x_hbm, tab_hbm, out_hbm, tab_v, tab_r, x_v, out_v3,
                sem0, sem1, sem2):
    wid = lax.axis_index("s") * NC + lax.axis_index("c")
    b0 = wid * BW

    pltpu.sync_copy(tab_hbm, tab_v)
    lane = lax.iota(jnp.int32, 16)
    tl = [tab_v[pl.ds(16 * m, 16)] for m in range(5 * EMB // 16)]

    def repl(l, carry):
        for m in range(5 * EMB // 16):
            plsc.store_scatter(tab_r, [l * ROFF + 16 * m + lane], tl[m])
        return carry

    lax.fori_loop(0, 16, repl, 0)

    lane_off = lane * ROFF

    sems = (sem0, sem1, sem2)

    def build(row, slot):
        @plsc.parallel_loop(0, BGRP, step=1, unroll=1)
        def bgrp(bb):
            boff = pl.multiple_of(bb * 16, 16)
            idxv = x_v[row, pl.ds(boff, 16)]
            addr = lane_off + idxv * EMB
            for e in range(EMB):
                out_v3[slot, e, pl.ds(boff, 16)] = plsc.load_gather(
                    tab_r, [addr + e])

    def step(s, carry):
        rem8 = lax.rem(s, SBLK)
        s0 = pl.multiple_of(s - rem8, SBLK)

        @pl.when(rem8 == 0)
        def _():
            pltpu.sync_copy(x_hbm.at[pl.ds(s0, SBLK), pl.ds(b0, BW)], x_v)

        rem3 = lax.rem(s, 3)
        for q in range(3):
            @pl.when((rem3 == q) & (s >= 3))
            def _():
                pltpu.make_async_copy(
                    out_v3.at[pl.ds(q, 1)],
                    out_hbm.at[pl.ds(s - 3, 1), :, pl.ds(b0, BW)],
                    sems[q]).wait()

        build(rem8, rem3)
        for q in range(3):
            @pl.when(rem3 == q)
            def _():
                pltpu.make_async_copy(
                    out_v3.at[pl.ds(q, 1)],
                    out_hbm.at[pl.ds(s, 1), :, pl.ds(b0, BW)],
                    sems[q]).start()
        return carry

    lax.fori_loop(0, SEQ, step, 0)

    for s in range(SEQ - 3, SEQ):
        q = s % 3
        pltpu.make_async_copy(
            out_v3.at[pl.ds(q, 1)],
            out_hbm.at[pl.ds(s, 1), :, pl.ds(b0, BW)], sems[q]).wait()


def kernel(x, connectivity_embedding):
    xt = x.T
    tab1d = connectivity_embedding.reshape(-1)
    out_t = _emb_lookup(xt, tab1d)
    return lax.transpose(out_t, (2, 0, 1))
